# baseline (speedup 1.0000x reference)
import functools

import jax
import jax.numpy as jnp
from jax import lax
from jax.experimental import pallas as pl
from jax.experimental.pallas import tpu as pltpu
from jax.experimental.pallas import tpu_sc as plsc

_BATCH = 16384
_DIM = 64
_GAMMA = 12.0
_NW = 32
_BPW = _BATCH // _NW
_CHUNK = 128
_NCHUNK = _BPW // _CHUNK
_GROUPS = _CHUNK // 16


def _tec_body(heads_h, rels_h, tails_h, nheads_h, ntails_h, ent_h, rel_h,
              out_h,
              h_idx, r_idx, t_idx, nh_idx, nt_idx,
              h_rows, r_rows, t_rows, nh_rows, nt_rows,
              out_stage, sems):
    wid = lax.axis_index("s") * 2 + lax.axis_index("c")
    base = wid * _BPW

    pltpu.sync_copy(heads_h.at[pl.ds(base, _BPW)], h_idx)
    pltpu.sync_copy(rels_h.at[pl.ds(base, _BPW)], r_idx)
    pltpu.sync_copy(tails_h.at[pl.ds(base, _BPW)], t_idx)
    pltpu.sync_copy(nheads_h.at[pl.ds(base, _BPW)], nh_idx)
    pltpu.sync_copy(ntails_h.at[pl.ds(base, _BPW)], nt_idx)

    def start_chunk(c, par):
        off = c * _CHUNK
        sem = sems.at[par]
        return [
            pltpu.async_copy(ent_h.at[h_idx.at[pl.ds(off, _CHUNK)]],
                             h_rows.at[par], sem),
            pltpu.async_copy(rel_h.at[r_idx.at[pl.ds(off, _CHUNK)]],
                             r_rows.at[par], sem),
            pltpu.async_copy(ent_h.at[t_idx.at[pl.ds(off, _CHUNK)]],
                             t_rows.at[par], sem),
            pltpu.async_copy(ent_h.at[nh_idx.at[pl.ds(off, _CHUNK)]],
                             nh_rows.at[par], sem),
            pltpu.async_copy(ent_h.at[nt_idx.at[pl.ds(off, _CHUNK)]],
                             nt_rows.at[par], sem),
        ]

    s_loss = jnp.float32(0.0)
    s_pos = jnp.float32(0.0)
    s_neg = jnp.float32(0.0)

    descs = [None, None]
    descs[0] = start_chunk(0, 0)

    for c in range(_NCHUNK):
        par = c % 2
        if c + 1 < _NCHUNK:
            descs[(c + 1) % 2] = start_chunk(c + 1, (c + 1) % 2)
        for d in descs[par]:
            d.wait()

        hb, rb, tb = h_rows.at[par], r_rows.at[par], t_rows.at[par]
        nhb, ntb = nh_rows.at[par], nt_rows.at[par]

        def elem_body(e, carry, hb=hb, rb=rb, tb=tb, nhb=nhb, ntb=ntb):
            ls, ps, ns = carry
            accp = accn = None
            for q in range(_DIM // 16):
                sl = pl.ds(q * 16, 16)
                h = hb[e, sl]
                r = rb[e, sl]
                t = tb[e, sl]
                nh = nhb[e, sl]
                nt = ntb[e, sl]
                pd = jnp.abs(h + r - t)
                nd = jnp.abs(nh + r - nt)
                accp = pd if q == 0 else accp + pd
                accn = nd if q == 0 else accn + nd
            pe = jnp.sum(accp)
            ne = jnp.sum(accn)
            return (ls + jnp.maximum(_GAMMA + pe - ne, 0.0),
                    ps + pe, ns + ne)

        s_loss, s_pos, s_neg = lax.fori_loop(
            0, _CHUNK, elem_body, (s_loss, s_pos, s_neg), unroll=4)

    out_stage[0, :] = jnp.full((16,), 0.0, jnp.float32) + s_loss
    out_stage[1, :] = jnp.full((16,), 0.0, jnp.float32) + s_pos
    out_stage[2, :] = jnp.full((16,), 0.0, jnp.float32) + s_neg
    pltpu.sync_copy(out_stage, out_h.at[wid])


@jax.jit
def _transe_sc(heads, relations, tails, negative_heads, negative_tails,
               entity_emb, relation_emb):
    mesh = plsc.VectorSubcoreMesh(core_axis_name="c", subcore_axis_name="s")
    partials = pl.kernel(
        _tec_body,
        out_type=jax.ShapeDtypeStruct((_NW, 3, 16), jnp.float32),
        mesh=mesh,
        compiler_params=pltpu.CompilerParams(needs_layout_passes=False,
                                             use_tc_tiling_on_sc=False),
        scratch_types=[
            pltpu.VMEM((_BPW,), jnp.int32),
            pltpu.VMEM((_BPW,), jnp.int32),
            pltpu.VMEM((_BPW,), jnp.int32),
            pltpu.VMEM((_BPW,), jnp.int32),
            pltpu.VMEM((_BPW,), jnp.int32),
            pltpu.VMEM((2, _CHUNK, _DIM), jnp.float32),
            pltpu.VMEM((2, _CHUNK, _DIM), jnp.float32),
            pltpu.VMEM((2, _CHUNK, _DIM), jnp.float32),
            pltpu.VMEM((2, _CHUNK, _DIM), jnp.float32),
            pltpu.VMEM((2, _CHUNK, _DIM), jnp.float32),
            pltpu.VMEM((3, 16), jnp.float32),
            pltpu.SemaphoreType.DMA((2,)),
        ],
    )(heads, relations, tails, negative_heads, negative_tails,
      entity_emb, relation_emb)
    sums = jnp.sum(partials[:, :, 0], axis=0)
    inv_b = 1.0 / _BATCH
    return sums[0] * inv_b, sums[1] * inv_b, sums[2] * inv_b


def kernel(heads, relations, tails, negative_heads, negative_tails,
           entity_emb, relation_emb):
    return _transe_sc(heads.astype(jnp.int32), relations.astype(jnp.int32),
                      tails.astype(jnp.int32),
                      negative_heads.astype(jnp.int32),
                      negative_tails.astype(jnp.int32),
                      entity_emb, relation_emb)
